# baseline (speedup 1.0000x reference)
import functools

import jax
import jax.numpy as jnp
from jax import lax
from jax.experimental import pallas as pl
from jax.experimental.pallas import tpu as pltpu
from jax.experimental.pallas import tpu_sc as plsc

MAX_LEN = 2048
BATCH = 16
NUM_AA = 21
NUM_CORES = 2
NUM_SUBCORES = 16
B_HALF = BATCH // 2
T_CHUNK = 128
N_TCHUNK = MAX_LEN // T_CHUNK

_mesh = plsc.VectorSubcoreMesh(core_axis_name="c", subcore_axis_name="s")


@functools.partial(
    pl.kernel,
    mesh=_mesh,
    out_type=jax.ShapeDtypeStruct((NUM_AA, BATCH, MAX_LEN), jnp.float32),
    scratch_types=[
        pltpu.VMEM((T_CHUNK, BATCH), jnp.int32),
        pltpu.VMEM((BATCH,), jnp.int32),
        pltpu.VMEM((NUM_AA, B_HALF, T_CHUNK), jnp.float32),
        pltpu.SemaphoreType.DMA,
    ],
    compiler_params=pltpu.CompilerParams(
        needs_layout_passes=False,
        skip_device_barrier=True,
        disable_bounds_checks=True,
        disable_semaphore_checks=True,
    ),
)
def _onehot_sc(data_hbm, len_hbm, out_hbm, data_v, len_v, out_v, sem):
    wid = lax.axis_index("s") * NUM_CORES + lax.axis_index("c")
    h = wid % 2
    tc = wid // 2
    t1 = tc * T_CHUNK
    b0 = h * B_HALF
    cp_data = pltpu.async_copy(data_hbm.at[pl.ds(t1, T_CHUNK)], data_v, sem)
    cp_len = pltpu.async_copy(len_hbm, len_v, sem)

    lanes = lax.iota(jnp.int32, 16)
    zero16 = jnp.zeros((16,), jnp.float32)
    one16 = jnp.ones((16,), jnp.float32)

    def zero_body(a, carry):
        for b in range(B_HALF):
            for j in range(T_CHUNK // 16):
                out_v[a, b, pl.ds(j * 16, 16)] = zero16
        return carry

    lax.fori_loop(0, NUM_AA, zero_body, 0)
    cp_data.wait()
    cp_len.wait()

    bloc = lanes & 7
    bvec = bloc + b0
    sel8 = lanes >> 3
    lens_g = plsc.load_gather(len_v, [bvec])

    def body(i, carry):
        tvec = jnp.full((16,), 2 * i, jnp.int32) + sel8
        row = plsc.load_gather(data_v, [tvec, bvec])
        mask = (t1 + tvec) < lens_g
        plsc.store_scatter(out_v, [row, bloc, tvec], one16, mask=mask)
        return carry

    lax.fori_loop(0, T_CHUNK // 2, body, 0)
    pltpu.sync_copy(out_v, out_hbm.at[:, pl.ds(b0, B_HALF), pl.ds(t1, T_CHUNK)])


def kernel(data, lengths, embed_init):
    del embed_init
    out = _onehot_sc(data, lengths)
    return jnp.transpose(out, (2, 1, 0))


# speedup vs baseline: 1.0066x; 1.0066x over previous
import functools

import jax
import jax.numpy as jnp
from jax import lax
from jax.experimental import pallas as pl
from jax.experimental.pallas import tpu as pltpu
from jax.experimental.pallas import tpu_sc as plsc

MAX_LEN = 2048
BATCH = 16
NUM_AA = 21
NUM_CORES = 2
NUM_SUBCORES = 16
B_HALF = BATCH // 2
T_CHUNK = 128
N_TCHUNK = MAX_LEN // T_CHUNK

_mesh = plsc.VectorSubcoreMesh(core_axis_name="c", subcore_axis_name="s")


@functools.partial(
    pl.kernel,
    mesh=_mesh,
    out_type=jax.ShapeDtypeStruct((NUM_AA, BATCH, MAX_LEN), jnp.float32),
    scratch_types=[
        pltpu.VMEM((T_CHUNK, BATCH), jnp.int32),
        pltpu.VMEM((BATCH,), jnp.int32),
        pltpu.VMEM((NUM_AA, B_HALF, T_CHUNK), jnp.float32),
        pltpu.SemaphoreType.DMA,
    ],
    compiler_params=pltpu.CompilerParams(needs_layout_passes=False),
)
def _onehot_sc(data_hbm, len_hbm, out_hbm, data_v, len_v, out_v, sem):
    wid = lax.axis_index("s") * NUM_CORES + lax.axis_index("c")
    h = wid % 2
    tc = wid // 2
    t1 = tc * T_CHUNK
    b0 = h * B_HALF
    cp_data = pltpu.async_copy(data_hbm.at[pl.ds(t1, T_CHUNK)], data_v, sem)
    cp_len = pltpu.async_copy(len_hbm, len_v, sem)

    lanes = lax.iota(jnp.int32, 16)
    zero16 = jnp.zeros((16,), jnp.float32)
    one16 = jnp.ones((16,), jnp.float32)

    def zero_body(a, carry):
        for b in range(B_HALF):
            for j in range(T_CHUNK // 16):
                out_v[a, b, pl.ds(j * 16, 16)] = zero16
        return carry

    lax.fori_loop(0, NUM_AA, zero_body, 0)
    cp_data.wait()
    cp_len.wait()

    bloc = lanes & 7
    bvec = bloc + b0
    sel8 = lanes >> 3
    lens_g = plsc.load_gather(len_v, [bvec])

    def body(i, carry):
        for u in range(2):
            tvec = jnp.full((16,), 4 * i + 2 * u, jnp.int32) + sel8
            row = plsc.load_gather(data_v, [tvec, bvec])
            mask = (t1 + tvec) < lens_g
            plsc.store_scatter(out_v, [row, bloc, tvec], one16, mask=mask)
        return carry

    lax.fori_loop(0, T_CHUNK // 4, body, 0)
    pltpu.sync_copy(out_v, out_hbm.at[:, pl.ds(b0, B_HALF), pl.ds(t1, T_CHUNK)])


def kernel(data, lengths, embed_init):
    del embed_init
    out = _onehot_sc(data, lengths)
    return jnp.transpose(out, (2, 1, 0))
